# baseline (speedup 1.0000x reference)
import jax
import jax.numpy as jnp
from jax import lax
from jax.experimental import pallas as pl
from jax.experimental.pallas import tpu as pltpu
from jax.experimental.pallas import tpu_sc as plsc

N = 10000
E = 320000
DF = 128
DE = 16
DO = 128

NS = 16
EPW = E // NS
CH = 40
NCH = EPW // CH
NP = 10240
RPT = NP // NS
CR = NP // 128


def _xa_body(x_ref, w_ref, o_ref):
    o_ref[...] = jnp.dot(x_ref[...], w_ref[...],
                         preferred_element_type=jnp.float32)


def _ep_body(ea_ref, w_ref, b_ref, o_ref):
    o_ref[...] = jnp.dot(ea_ref[...], w_ref[...],
                         preferred_element_type=jnp.float32) + b_ref[...]


def _out_body(p_ref, c_ref, w_ref, b_ref, o_ref):
    o_ref[...] = (jnp.dot(p_ref[...], w_ref[...],
                          preferred_element_type=jnp.float32)
                  + c_ref[...] * b_ref[...])


def _sc_body(xa_hbm, ep_hbm, idx_hbm, part_hbm, cnt_hbm,
             ix0, ix1, ix2, ix3, ga0, ga1, ep0, ep1, h0, h1,
             cnt_v, idn_v, acc_s, cnt_s,
             sI0, sI1, sI2, sI3, sG0, sG1, sE0, sE1, sS0, sS1):
    sid = lax.axis_index("s")
    base = sid * EPW

    ix = [ix0, ix1, ix2, ix3]
    sI = [sI0, sI1, sI2, sI3]
    ga = [ga0, ga1]
    sG = [sG0, sG1]
    epb = [ep0, ep1]
    sE = [sE0, sE1]
    hb = [h0, h1]
    sS = [sS0, sS1]

    zvec = jnp.zeros((16,), jnp.float32)
    lanes = lax.iota(jnp.int32, 16)

    def hrow(r, carry):
        for c in range(DO // 16):
            h0[r, pl.ds(c * 16, 16)] = zvec
        return carry

    lax.fori_loop(0, CH, hrow, 0)

    def crow(r, carry):
        for c in range(DO // 16):
            cnt_v[r, pl.ds(c * 16, 16)] = zvec
        return carry

    lax.fori_loop(0, CR, crow, 0)

    for g in range(CR // 16):
        idn_v[0, pl.ds(g * 16, 16)] = lanes + g * 16

    for k in range(RPT // CH):
        pltpu.sync_copy(h0, acc_s.at[pl.ds(sid * RPT + k * CH, CH)])

    @pl.when(sid == 0)
    def _():
        for k in range(CR // CH):
            pltpu.sync_copy(h0, cnt_s.at[pl.ds(k * CH, CH)])

    plsc.subcore_barrier()

    def idx_start(c, k):
        pltpu.async_copy(idx_hbm.at[sid, c], ix[k], sI[k])

    def idx_wait(c, k):
        pltpu.make_async_copy(idx_hbm.at[sid, c], ix[k], sI[k]).wait()

    def ge_start(c, k, p):
        pltpu.async_copy(xa_hbm.at[ix[k].at[0]], ga[p], sG[p])
        pltpu.async_copy(ep_hbm.at[pl.ds(base + c * CH, CH)], epb[p], sE[p])

    def ge_wait(c, k, p):
        pltpu.make_async_copy(xa_hbm.at[ix[k].at[0]], ga[p], sG[p]).wait()
        pltpu.make_async_copy(ep_hbm.at[pl.ds(base + c * CH, CH)],
                              epb[p], sE[p]).wait()

    def sc_start(k, p):
        pltpu.async_copy(hb[p], acc_s.at[ix[k].at[1]], sS[p], add=True)

    def sc_wait(k, p):
        pltpu.make_async_copy(hb[p], acc_s.at[ix[k].at[1]], sS[p]).wait()

    idx_start(0, 0)
    idx_start(1, 1)
    idx_wait(0, 0)
    ge_start(0, 0, 0)

    def step(t, carry):
        for k in range(4):
            c = t * 4 + k
            kn = (k + 1) % 4
            kp = (k + 2) % 4
            p = k % 2

            @pl.when(c + 1 < NCH)
            def _():
                idx_wait(c + 1, kn)
                ge_start(c + 1, kn, 1 - p)

            ge_wait(c, k, p)

            @pl.when(c >= 2)
            def _():
                sc_wait(kp, p)

            @pl.when(c + 2 < NCH)
            def _():
                idx_start(c + 2, kp)

            def row(r, c2):
                for cc in range(DO // 16):
                    s = pl.ds(cc * 16, 16)
                    hb[p][r, s] = jnp.maximum(ga[p][r, s] + epb[p][r, s],
                                              jnp.float32(0.0))
                return c2

            lax.fori_loop(0, CH, row, 0)

            for g, (goff, l0) in enumerate([(0, 0), (16, 0), (24, 8)]):
                dvec = ix[k][1, pl.ds(goff, 16)]
                for l in range(l0, 16):
                    d = dvec[l]
                    rw = lax.shift_right_logical(d, 7)
                    cl = jnp.bitwise_and(d, 127)
                    off = jnp.bitwise_and(cl, 112)
                    seg = pl.ds(off, 16)
                    onehot = jnp.where(lanes + off == cl,
                                       jnp.float32(1.0), jnp.float32(0.0))
                    cnt_v[rw, seg] = cnt_v[rw, seg] + onehot

            sc_start(k, p)
        return carry

    lax.fori_loop(0, NCH // 4, step, 0)

    sc_wait(2, 0)
    sc_wait(3, 1)

    pltpu.sync_copy(cnt_v, cnt_s.at[idn_v.at[0]], add=True)

    plsc.subcore_barrier()
    pltpu.sync_copy(acc_s.at[pl.ds(sid * RPT, RPT)],
                    part_hbm.at[pl.ds(sid * RPT, RPT)])

    @pl.when(sid == 0)
    def _():
        pltpu.sync_copy(cnt_s, cnt_hbm)


def kernel(x, edge_index, edge_attr, W1, b1, W2, b2):
    x = x.astype(jnp.float32)
    src = edge_index[0].astype(jnp.int32).reshape(NS, NCH, 1, CH)
    dst = edge_index[1].astype(jnp.int32).reshape(NS, NCH, 1, CH)
    idx = jnp.concatenate([src, dst], axis=2)

    W1a = W1[:DF]
    W1b = W1[DF:]

    xa = pl.pallas_call(
        _xa_body,
        grid=(10,),
        in_specs=[pl.BlockSpec((N // 10, DF), lambda i: (i, 0)),
                  pl.BlockSpec((DF, DO), lambda i: (0, 0))],
        out_specs=pl.BlockSpec((N // 10, DO), lambda i: (i, 0)),
        out_shape=jax.ShapeDtypeStruct((N, DO), jnp.float32),
    )(x, W1a)

    EB = 3200
    ep = pl.pallas_call(
        _ep_body,
        grid=(E // EB,),
        in_specs=[pl.BlockSpec((EB, DE), lambda i: (i, 0)),
                  pl.BlockSpec((DE, DO), lambda i: (0, 0)),
                  pl.BlockSpec((1, DO), lambda i: (0, 0))],
        out_specs=pl.BlockSpec((EB, DO), lambda i: (i, 0)),
        out_shape=jax.ShapeDtypeStruct((E, DO), jnp.float32),
    )(edge_attr.astype(jnp.float32), W1b, b1.reshape(1, DO))

    mesh = plsc.VectorSubcoreMesh(core_axis_name="c", subcore_axis_name="s",
                                  num_cores=1)
    part, cnt = pl.kernel(
        _sc_body,
        out_type=(jax.ShapeDtypeStruct((NP, DO), jnp.float32),
                  jax.ShapeDtypeStruct((CR, DO), jnp.float32)),
        mesh=mesh,
        scratch_types=(
            [pltpu.VMEM((2, CH), jnp.int32)] * 4
            + [pltpu.VMEM((CH, DO), jnp.float32)] * 2
            + [pltpu.VMEM((CH, DO), jnp.float32)] * 2
            + [pltpu.VMEM((CH, DO), jnp.float32)] * 2
            + [pltpu.VMEM((CR, DO), jnp.float32),
               pltpu.VMEM((1, CR), jnp.int32),
               pltpu.VMEM_SHARED((NP, DO), jnp.float32),
               pltpu.VMEM_SHARED((CR, DO), jnp.float32)]
            + [pltpu.SemaphoreType.DMA] * 10
        ),
    )(xa, ep, idx)

    cnt1d = cnt.reshape(NP, 1)

    out = pl.pallas_call(
        _out_body,
        grid=(10,),
        in_specs=[pl.BlockSpec((NP // 10, DO), lambda i: (i, 0)),
                  pl.BlockSpec((NP // 10, 1), lambda i: (i, 0)),
                  pl.BlockSpec((DO, DO), lambda i: (0, 0)),
                  pl.BlockSpec((1, DO), lambda i: (0, 0))],
        out_specs=pl.BlockSpec((NP // 10, DO), lambda i: (i, 0)),
        out_shape=jax.ShapeDtypeStruct((NP, DO), jnp.float32),
    )(part, cnt1d, W2, b2.reshape(1, DO))
    return out[:N]


# speedup vs baseline: 1.0064x; 1.0064x over previous
import jax
import jax.numpy as jnp
from jax import lax
from jax.experimental import pallas as pl
from jax.experimental.pallas import tpu as pltpu
from jax.experimental.pallas import tpu_sc as plsc

N = 10000
E = 320000
DF = 128
DE = 16
DO = 128

NS = 16
EPW = E // NS
CH = 40
NCH = EPW // CH
NP = 10240
RPT = NP // NS
CR = NP // 128


def _xa_body(x_ref, w_ref, o_ref):
    o_ref[...] = jnp.dot(x_ref[...], w_ref[...],
                         preferred_element_type=jnp.float32)


def _ep_body(ea_ref, w_ref, b_ref, o_ref):
    o_ref[...] = jnp.dot(ea_ref[...], w_ref[...],
                         preferred_element_type=jnp.float32) + b_ref[...]


def _out_body(p_ref, c_ref, w_ref, b_ref, o_ref):
    o_ref[...] = (jnp.dot(p_ref[...], w_ref[...],
                          preferred_element_type=jnp.float32)
                  + c_ref[...] * b_ref[...])


def _sc_body(xa_hbm, ep_hbm, idx_hbm, part_hbm, cnt_hbm,
             ix0, ix1, ix2, ix3, ga0, ga1, ep0, ep1, h0, h1,
             cnt_v, idn_v, acc_s, cnt_s,
             sI0, sI1, sI2, sI3, sG0, sG1, sE0, sE1, sS0, sS1):
    sid = lax.axis_index("s")
    base = sid * EPW

    ix = [ix0, ix1, ix2, ix3]
    sI = [sI0, sI1, sI2, sI3]
    ga = [ga0, ga1]
    sG = [sG0, sG1]
    epb = [ep0, ep1]
    sE = [sE0, sE1]
    hb = [h0, h1]
    sS = [sS0, sS1]

    zvec = jnp.zeros((16,), jnp.float32)
    lanes = lax.iota(jnp.int32, 16)

    def hrow(r, carry):
        for c in range(DO // 16):
            h0[r, pl.ds(c * 16, 16)] = zvec
        return carry

    lax.fori_loop(0, CH, hrow, 0)

    def crow(r, carry):
        for c in range(DO // 16):
            cnt_v[r, pl.ds(c * 16, 16)] = zvec
        return carry

    lax.fori_loop(0, CR, crow, 0)

    for g in range(CR // 16):
        idn_v[0, pl.ds(g * 16, 16)] = lanes + g * 16

    for k in range(RPT // CH):
        pltpu.sync_copy(h0, acc_s.at[pl.ds(sid * RPT + k * CH, CH)])

    @pl.when(sid == 0)
    def _():
        for k in range(CR // CH):
            pltpu.sync_copy(h0, cnt_s.at[pl.ds(k * CH, CH)])

    plsc.subcore_barrier()

    def idx_start(c, k):
        pltpu.async_copy(idx_hbm.at[sid, c], ix[k], sI[k])

    def idx_wait(c, k):
        pltpu.make_async_copy(idx_hbm.at[sid, c], ix[k], sI[k]).wait()

    def ge_start(c, k, p):
        pltpu.async_copy(xa_hbm.at[ix[k].at[0]], ga[p], sG[p])
        pltpu.async_copy(ep_hbm.at[pl.ds(base + c * CH, CH)], epb[p], sE[p])

    def ge_wait(c, k, p):
        pltpu.make_async_copy(xa_hbm.at[ix[k].at[0]], ga[p], sG[p]).wait()
        pltpu.make_async_copy(ep_hbm.at[pl.ds(base + c * CH, CH)],
                              epb[p], sE[p]).wait()

    def sc_start(k, p):
        pltpu.async_copy(hb[p], acc_s.at[ix[k].at[1]], sS[p], add=True)

    def sc_wait(k, p):
        pltpu.make_async_copy(hb[p], acc_s.at[ix[k].at[1]], sS[p]).wait()

    idx_start(0, 0)
    idx_start(1, 1)
    idx_wait(0, 0)
    ge_start(0, 0, 0)

    def step(t, carry):
        for k in range(4):
            c = t * 4 + k
            kn = (k + 1) % 4
            kp = (k + 2) % 4
            p = k % 2

            @pl.when(c + 1 < NCH)
            def _():
                idx_wait(c + 1, kn)
                ge_start(c + 1, kn, 1 - p)

            ge_wait(c, k, p)

            @pl.when(c >= 2)
            def _():
                sc_wait(kp, p)

            @pl.when(c + 2 < NCH)
            def _():
                idx_start(c + 2, kp)

            def row(r, c2):
                for cc in range(DO // 16):
                    s = pl.ds(cc * 16, 16)
                    hb[p][r, s] = jnp.maximum(ga[p][r, s] + epb[p][r, s],
                                              jnp.float32(0.0))
                return c2

            lax.fori_loop(0, CH, row, 0)

            for g, (goff, l0) in enumerate([(0, 0), (16, 0), (24, 8)]):
                dvec = ix[k][1, pl.ds(goff, 16)]
                for l in range(l0, 16):
                    d = dvec[l]
                    rw = lax.shift_right_logical(d, 7)
                    cl = jnp.bitwise_and(d, 127)
                    off = jnp.bitwise_and(cl, 112)
                    seg = pl.ds(off, 16)
                    onehot = jnp.where(lanes + off == cl,
                                       jnp.float32(1.0), jnp.float32(0.0))
                    cnt_v[rw, seg] = cnt_v[rw, seg] + onehot

            sc_start(k, p)
        return carry

    lax.fori_loop(0, NCH // 4, step, 0)

    sc_wait(2, 0)
    sc_wait(3, 1)

    pltpu.sync_copy(cnt_v, cnt_s.at[idn_v.at[0]], add=True)

    plsc.subcore_barrier()
    pltpu.sync_copy(acc_s.at[pl.ds(sid * RPT, RPT)],
                    part_hbm.at[pl.ds(sid * RPT, RPT)])

    @pl.when(sid == 0)
    def _():
        pltpu.sync_copy(cnt_s, cnt_hbm)


def kernel(x, edge_index, edge_attr, W1, b1, W2, b2):
    x = x.astype(jnp.float32)
    src = edge_index[0].astype(jnp.int32).reshape(NS, NCH, 1, CH)
    dst = edge_index[1].astype(jnp.int32).reshape(NS, NCH, 1, CH)
    idx = jnp.concatenate([src, dst], axis=2)

    W1a = W1[:DF]
    W1b = W1[DF:]

    xa = pl.pallas_call(
        _xa_body,
        grid=(10,),
        in_specs=[pl.BlockSpec((N // 10, DF), lambda i: (i, 0)),
                  pl.BlockSpec((DF, DO), lambda i: (0, 0))],
        out_specs=pl.BlockSpec((N // 10, DO), lambda i: (i, 0)),
        out_shape=jax.ShapeDtypeStruct((N, DO), jnp.float32),
    )(x, W1a)

    EB = 3200
    ep = pl.pallas_call(
        _ep_body,
        grid=(E // EB,),
        in_specs=[pl.BlockSpec((EB, DE), lambda i: (i, 0)),
                  pl.BlockSpec((DE, DO), lambda i: (0, 0)),
                  pl.BlockSpec((1, DO), lambda i: (0, 0))],
        out_specs=pl.BlockSpec((EB, DO), lambda i: (i, 0)),
        out_shape=jax.ShapeDtypeStruct((E, DO), jnp.float32),
    )(edge_attr.astype(jnp.float32), W1b, b1.reshape(1, DO))

    mesh = plsc.VectorSubcoreMesh(core_axis_name="c", subcore_axis_name="s",
                                  num_cores=1)
    part, cnt = pl.kernel(
        _sc_body,
        out_type=(jax.ShapeDtypeStruct((NP, DO), jnp.float32),
                  jax.ShapeDtypeStruct((CR, DO), jnp.float32)),
        mesh=mesh,
        scratch_types=(
            [pltpu.VMEM((2, CH), jnp.int32)] * 4
            + [pltpu.VMEM((CH, DO), jnp.float32)] * 2
            + [pltpu.VMEM((CH, DO), jnp.float32)] * 2
            + [pltpu.VMEM((CH, DO), jnp.float32)] * 2
            + [pltpu.VMEM((CR, DO), jnp.float32),
               pltpu.VMEM((1, CR), jnp.int32),
               pltpu.VMEM_SHARED((NP, DO), jnp.float32),
               pltpu.VMEM_SHARED((CR, DO), jnp.float32)]
            + [pltpu.SemaphoreType.DMA] * 10
        ),
    )(xa, ep, idx)

    cnt1d = cnt.reshape(NP, 1)

    out = pl.pallas_call(
        _out_body,
        grid=(10,),
        in_specs=[pl.BlockSpec((N // 10, DO), lambda i: (i, 0)),
                  pl.BlockSpec((N // 10, 1), lambda i: (i, 0)),
                  pl.BlockSpec((DO, DO), lambda i: (0, 0)),
                  pl.BlockSpec((1, DO), lambda i: (0, 0))],
        out_specs=pl.BlockSpec((N // 10, DO), lambda i: (i, 0)),
        out_shape=jax.ShapeDtypeStruct((N, DO), jnp.float32),
    )(part, cnt1d, W2, b2.reshape(1, DO))
    return out
